# baseline (speedup 1.0000x reference)
import jax
import jax.numpy as jnp
from jax.experimental import pallas as pl
from jax.experimental.pallas import tpu as pltpu
from jax.experimental.pallas import tpu_sc as plsc

B, S, D, F, E = 2, 2048, 1024, 4096, 8
T = B * S
G = T // E
BT2 = 128
MW = 128
SCW = 128


def _router_kernel(x_ref, wu_ref, bu_ref, invp_ref, meta_ref, q_ref):
    z = jax.lax.dot_general(
        x_ref[...], wu_ref[...],
        (((1,), (0,)), ((), ())), preferred_element_type=jnp.float32)
    u = jax.nn.sigmoid(z + bu_ref[0, 0])
    meta_ref[...] = jnp.broadcast_to(u, (T, MW))
    n = jnp.clip(jnp.ceil(u * E), 1, E)
    n2 = n.reshape(G, E)
    n64 = jnp.repeat(n2, E, axis=1)
    v64 = (jax.lax.broadcasted_iota(jnp.int32, (G, E * E), 1) % E
           ).astype(jnp.float32)
    oh_eq = (n64 == v64 + 1).astype(jnp.float32)
    q64 = jnp.sum((n64 > v64).astype(jnp.float32), axis=0, keepdims=True)
    qs64 = jnp.sum((n64 > v64 + 1).astype(jnp.float32), axis=0, keepdims=True)
    ii = jax.lax.broadcasted_iota(jnp.int32, (G, G), 0)
    jj = jax.lax.broadcasted_iota(jnp.int32, (G, G), 1)
    tri = (jj < ii).astype(jnp.bfloat16)
    cum_eq = jax.lax.dot_general(
        tri, oh_eq.astype(jnp.bfloat16),
        (((1,), (0,)), ((), ())), preferred_element_type=jnp.float32)
    a64 = oh_eq * (qs64 + cum_eq)
    pos = jnp.sum(a64.reshape(G, E, E), axis=2)
    r2 = jax.lax.broadcasted_iota(jnp.int32, (G, E), 1)
    invp_ref[...] = pos.astype(jnp.int32) + r2 * G
    q_ref[...] = q64.astype(jnp.int32)


def _run_router(xb, wub, bu2):
    return pl.pallas_call(
        _router_kernel,
        grid=(1,),
        in_specs=[
            pl.BlockSpec((T, D), lambda i: (0, 0)),
            pl.BlockSpec((D, 1), lambda i: (0, 0)),
            pl.BlockSpec((1, 1), lambda i: (0, 0)),
        ],
        out_specs=[
            pl.BlockSpec((G, E), lambda i: (0, 0)),
            pl.BlockSpec((T, MW), lambda i: (0, 0)),
            pl.BlockSpec((1, E * E), lambda i: (0, 0)),
        ],
        out_shape=[
            jax.ShapeDtypeStruct((G, E), jnp.int32),
            jax.ShapeDtypeStruct((T, MW), jnp.float32),
            jax.ShapeDtypeStruct((1, E * E), jnp.int32),
        ],
    )(xb, wub, bu2)


def _sc_permute_one(src, idx, scatter):
    W = src.shape[1]
    vector_mesh = plsc.VectorSubcoreMesh(
        core_axis_name="core", subcore_axis_name="subcore")

    @pl.kernel(out_type=jax.ShapeDtypeStruct(src.shape, src.dtype),
               mesh=vector_mesh)
    def kperm(s_hbm, i_hbm, d_hbm):
        if scatter:
            def body(s_vmem, i_vmem):
                pltpu.sync_copy(s_vmem, d_hbm.at[i_vmem.at[0]])

            pltpu.emit_pipeline(
                body,
                grid=(T // SCW,),
                in_specs=[
                    pl.BlockSpec((SCW, W), lambda i: (i, 0)),
                    pl.BlockSpec((1, SCW), lambda i: (0, i)),
                ],
                out_specs=[],
                core_axis_name=("core", "subcore"),
                dimension_semantics=(pltpu.PARALLEL,),
            )(s_hbm, i_hbm)
        else:
            def body(i_vmem, o_vmem):
                pltpu.sync_copy(s_hbm.at[i_vmem.at[0]], o_vmem)

            pltpu.emit_pipeline(
                body,
                grid=(T // SCW,),
                in_specs=[pl.BlockSpec((1, SCW), lambda i: (0, i))],
                out_specs=[pl.BlockSpec((SCW, W), lambda i: (i, 0))],
                core_axis_name=("core", "subcore"),
                dimension_semantics=(pltpu.PARALLEL,),
            )(i_hbm, d_hbm)

    return kperm(src, idx)


def _sc_scatter(xb, meta, invp_flat):
    x_i32 = jax.lax.bitcast_convert_type(
        xb.reshape(T, D // 2, 2), jnp.int32)
    HW = D // 4
    xs_a = _sc_permute_one(x_i32[:, :HW], invp_flat, True)
    xs_b = _sc_permute_one(x_i32[:, HW:], invp_flat, True)
    ms = _sc_permute_one(meta, invp_flat, True)
    xs_i32 = jnp.concatenate([xs_a, xs_b], axis=1)
    xs = jax.lax.bitcast_convert_type(xs_i32, jnp.bfloat16).reshape(T, D)
    return xs, ms


def _sparse_ffn_kernel(q_ref, xs_ref, ms_ref, w1_ref, b1_ref, w2_ref, b2_ref,
                       out_ref):
    j = pl.program_id(0)
    p = pl.program_id(1)

    @pl.when(jnp.logical_and(j == 0, p == 0))
    def _():
        for blk in range(T // G):
            out_ref[pl.ds(blk * G, G), :] = jnp.zeros((G, D), jnp.float32)

    w1 = w1_ref[0]
    b1 = b1_ref[0]
    w2 = w2_ref[0]
    b2 = b2_ref[0]
    for r in range(E):
        k = (j + (E - r)) & (E - 1)
        kf = k.astype(jnp.float32)
        q_rk = q_ref[r * E + k]
        nb = (q_rk + BT2 - 1) // BT2

        def body(tb, _, r=r, kf=kf):
            row0 = r * G + tb * BT2
            rows = pl.ds(row0, BT2)
            u_col = ms_ref[rows, 0:1]
            n_col = jnp.clip(jnp.ceil(u_col * E), 1, E)
            c_col = jnp.where(n_col > kf, u_col / (kf + 1.0), 0.0)
            h = jax.lax.dot_general(
                xs_ref[rows, :], w1,
                (((1,), (0,)), ((), ())), preferred_element_type=jnp.float32)
            h = jnp.maximum(h + b1, 0.0)
            hw = (h * c_col).astype(jnp.bfloat16)
            y = jax.lax.dot_general(
                hw, w2,
                (((1,), (0,)), ((), ())), preferred_element_type=jnp.float32)
            y = jnp.where(p == 0, y + c_col * b2, y)
            out_ref[rows, :] += y
            return 0

        jax.lax.fori_loop(0, nb, body, 0)


def _run_sparse_ffn(q_flat, xs, ms, w1b, b1r, w2b, b2r):
    FH = F // 2
    grid_spec = pltpu.PrefetchScalarGridSpec(
        num_scalar_prefetch=1,
        grid=(E, 2),
        in_specs=[
            pl.BlockSpec((T, D), lambda j, p, q: (0, 0)),
            pl.BlockSpec((T, E), lambda j, p, q: (0, 0)),
            pl.BlockSpec((1, D, FH), lambda j, p, q: (j, 0, p)),
            pl.BlockSpec((1, 1, FH), lambda j, p, q: (j, 0, p)),
            pl.BlockSpec((1, FH, D), lambda j, p, q: (j, p, 0)),
            pl.BlockSpec((1, 1, D), lambda j, p, q: (j, 0, 0)),
        ],
        out_specs=pl.BlockSpec((T, D), lambda j, p, q: (0, 0)),
    )
    return pl.pallas_call(
        _sparse_ffn_kernel,
        grid_spec=grid_spec,
        out_shape=jax.ShapeDtypeStruct((T, D), jnp.float32),
        compiler_params=pltpu.CompilerParams(
            dimension_semantics=("arbitrary", "arbitrary"),
        ),
    )(q_flat, xs, ms[:, :E], w1b, b1r, w2b, b2r)


def _sc_gather(outs, invp_flat):
    QW = D // 4
    parts = [
        _sc_permute_one(outs[:, i * QW:(i + 1) * QW], invp_flat, False)
        for i in range(4)
    ]
    return jnp.concatenate(parts, axis=1)


@jax.jit
def kernel(x, W1, b1, W2, b2, Wu, bu):
    xb = x.reshape(T, D).astype(jnp.bfloat16)
    w1b = W1.astype(jnp.bfloat16)
    w2b = W2.astype(jnp.bfloat16)
    wub = Wu.astype(jnp.bfloat16)
    bu2 = bu.reshape(1, 1)
    b1r = b1.reshape(E, 1, F)
    b2r = b2.reshape(E, 1, D)

    invp, meta, q = _run_router(xb, wub, bu2)
    invp_flat = invp.reshape(1, T)
    xs, ms = _sc_scatter(xb, meta, invp_flat)
    outs = _run_sparse_ffn(q.reshape(E * E), xs, ms, w1b, b1r, w2b, b2r)
    out = _sc_gather(outs, invp_flat)
    return out.reshape(B, S, D)
